# Initial kernel scaffold; baseline (speedup 1.0000x reference)
import functools

import jax
import jax.numpy as jnp
from jax import lax
from jax.experimental import pallas as pl
from jax.experimental.pallas import tpu as pltpu
from jax.experimental.pallas import tpu_sc as plsc

N = 10000
E = 320000
D = 128
H = 128
C = 8
G = 64

TILES = 32
CHUNKS = 79
K = 128
EPAD = TILES * CHUNKS * K
NP = 10240
RPT = NP // 16

_mesh = plsc.VectorSubcoreMesh(core_axis_name="c", subcore_axis_name="s")


@functools.partial(
    pl.kernel,
    mesh=_mesh,
    out_type=jax.ShapeDtypeStruct((2, NP), jnp.float32),
    scratch_types=[
        pltpu.VMEM((CHUNKS, K), jnp.int32),
        pltpu.VMEM((CHUNKS, K), jnp.float32),
        pltpu.VMEM_SHARED((NP,), jnp.float32),
    ],
)
def _deg_sc(dst_hbm, ew_hbm, z1_hbm, out_hbm, dst_v, ew_v, acc):
    c = lax.axis_index("c")
    s = lax.axis_index("s")
    b = c * 16 + s
    r0 = pl.multiple_of(s * RPT, 8)
    pltpu.sync_copy(z1_hbm, acc.at[pl.ds(r0, RPT)])
    plsc.subcore_barrier()
    pltpu.sync_copy(dst_hbm.at[b], dst_v)
    pltpu.sync_copy(ew_hbm.at[b], ew_v)

    def chunk(j, carry):
        pltpu.sync_copy(ew_v.at[j], acc.at[dst_v.at[j]], add=True)
        return carry

    lax.fori_loop(0, CHUNKS, chunk, 0)
    plsc.subcore_barrier()
    pltpu.sync_copy(acc.at[pl.ds(r0, RPT)], out_hbm.at[c, pl.ds(r0, RPT)])


@functools.partial(
    pl.kernel,
    mesh=_mesh,
    out_type=jax.ShapeDtypeStruct((2, NP, H), jnp.float32),
    scratch_types=[
        pltpu.VMEM((CHUNKS, K), jnp.int32),
        pltpu.VMEM((CHUNKS, K), jnp.int32),
        pltpu.VMEM((CHUNKS, K), jnp.float32),
        pltpu.VMEM((K, H), jnp.float32),
        pltpu.VMEM_SHARED((NP, H), jnp.float32),
        pltpu.SemaphoreType.DMA,
    ],
)
def _agg_sc(hp_hbm, src_hbm, dst_hbm, ew_hbm, z2_hbm, out_hbm,
            src_v, dst_v, ew_v, rows_v, acc, sem):
    c = lax.axis_index("c")
    s = lax.axis_index("s")
    b = c * 16 + s
    r0 = pl.multiple_of(s * RPT, 8)
    pltpu.sync_copy(z2_hbm, acc.at[pl.ds(r0, RPT)])
    plsc.subcore_barrier()
    pltpu.sync_copy(src_hbm.at[b], src_v)
    pltpu.sync_copy(dst_hbm.at[b], dst_v)
    pltpu.sync_copy(ew_hbm.at[b], ew_v)

    def chunk(j, carry):
        pltpu.async_copy(hp_hbm.at[src_v.at[j]], rows_v, sem).wait()
        j16 = jnp.full((16,), 0, jnp.int32) + j

        def edge(e, ecarry):
            e16 = jnp.full((16,), 0, jnp.int32) + e
            w = plsc.load_gather(ew_v, [j16, e16])
            for cg in range(8):
                col = cg * 16 + lax.iota(jnp.int32, 16)
                r = plsc.load_gather(rows_v, [e16, col])
                plsc.store_scatter(rows_v, [e16, col], r * w)
            return ecarry

        lax.fori_loop(0, K, edge, 0)
        pltpu.sync_copy(rows_v, acc.at[dst_v.at[j]], add=True)
        return carry

    lax.fori_loop(0, CHUNKS, chunk, 0)
    plsc.subcore_barrier()
    pltpu.sync_copy(acc.at[pl.ds(r0, RPT)], out_hbm.at[c, pl.ds(r0, RPT)])


def _tc1_body(deg0_ref, deg1_ref, x_ref, w_ref, dis_ref, hp_ref):
    deg = 1.0 + deg0_ref[...] + deg1_ref[...]
    dis = jnp.where(deg > 0, lax.rsqrt(deg), 0.0)
    dis_ref[...] = dis
    h = jnp.dot(x_ref[...], w_ref[...], preferred_element_type=jnp.float32,
                precision=lax.Precision.HIGHEST)
    hp_ref[...] = h * dis


def _tc_mid_body(a0_ref, a1_ref, hp_ref, dis_ref, b_ref, w_ref, out_ref):
    dis = dis_ref[...]
    t = (a0_ref[...] + a1_ref[...] + hp_ref[...]) * dis + b_ref[...]
    o = jnp.maximum(t, 0.0)
    out_ref[...] = jnp.dot(o, w_ref[...], preferred_element_type=jnp.float32,
                           precision=lax.Precision.HIGHEST) * dis


def _tc_fin_body(a0_ref, a1_ref, hp_ref, dis_ref, b_ref, brow_ref, wl_ref,
                 bl_ref, out_ref):
    o3 = (a0_ref[...] + a1_ref[...] + hp_ref[...]) * dis_ref[...] + b_ref[...]
    gid = lax.broadcasted_iota(jnp.int32, (G, N), 0)
    oh = (gid == brow_ref[...]).astype(jnp.float32)
    sums = jnp.dot(oh, o3, preferred_element_type=jnp.float32,
                   precision=lax.Precision.HIGHEST)
    cnt = jnp.dot(oh, jnp.ones((N, 1), jnp.float32),
                  preferred_element_type=jnp.float32,
                  precision=lax.Precision.HIGHEST)
    pooled = sums / jnp.maximum(cnt, 1.0)
    out_ref[...] = jnp.dot(pooled, wl_ref[...],
                           preferred_element_type=jnp.float32,
                           precision=lax.Precision.HIGHEST) + bl_ref[...]


def _pc(body, out_shapes):
    return pl.pallas_call(body, out_shape=out_shapes)


def kernel(x, edge_index, edge_attr, batch, W1, b1, W2, b2, W3, b3, Wl, bl):
    pad = EPAD - E
    src3 = jnp.concatenate(
        [edge_index[0], jnp.zeros((pad,), jnp.int32)]).reshape(TILES, CHUNKS, K)
    dst3 = jnp.concatenate(
        [edge_index[1], jnp.zeros((pad,), jnp.int32)]).reshape(TILES, CHUNKS, K)
    ew3 = jnp.concatenate(
        [edge_attr, jnp.zeros((pad,), jnp.float32)]).reshape(TILES, CHUNKS, K)
    z1 = jnp.zeros((RPT,), jnp.float32)
    z2 = jnp.zeros((RPT, H), jnp.float32)
    brow = batch[None, :]

    deg2 = _deg_sc(dst3, ew3, z1)
    deg0 = deg2[0, :N, None]
    deg1 = deg2[1, :N, None]
    dis, hp1 = _pc(_tc1_body, [
        jax.ShapeDtypeStruct((N, 1), jnp.float32),
        jax.ShapeDtypeStruct((N, H), jnp.float32),
    ])(deg0, deg1, x, W1)

    a1 = _agg_sc(hp1, src3, dst3, ew3, z2)
    hp2 = _pc(_tc_mid_body, jax.ShapeDtypeStruct((N, H), jnp.float32))(
        a1[0, :N], a1[1, :N], hp1, dis, b1[None, :], W2)

    a2 = _agg_sc(hp2, src3, dst3, ew3, z2)
    hp3 = _pc(_tc_mid_body, jax.ShapeDtypeStruct((N, H), jnp.float32))(
        a2[0, :N], a2[1, :N], hp2, dis, b2[None, :], W3)

    a3 = _agg_sc(hp3, src3, dst3, ew3, z2)
    out = _pc(_tc_fin_body, jax.ShapeDtypeStruct((G, C), jnp.float32))(
        a3[0, :N], a3[1, :N], hp3, dis, b3[None, :], brow, Wl, bl[None, :])
    return out


# speedup vs baseline: 9.4071x; 9.4071x over previous
import functools

import jax
import jax.numpy as jnp
from jax import lax
from jax.experimental import pallas as pl
from jax.experimental.pallas import tpu as pltpu
from jax.experimental.pallas import tpu_sc as plsc

N = 10000
E = 320000
D = 128
H = 128
C = 8
G = 64

TILES = 32
CHUNKS = 79
K = 128
EPAD = TILES * CHUNKS * K
NP = 10240
RPT = NP // 16

_mesh = plsc.VectorSubcoreMesh(core_axis_name="c", subcore_axis_name="s")


@functools.partial(
    pl.kernel,
    mesh=_mesh,
    out_type=jax.ShapeDtypeStruct((2, NP), jnp.float32),
    scratch_types=[
        pltpu.VMEM((CHUNKS, K), jnp.int32),
        pltpu.VMEM((CHUNKS, K), jnp.float32),
        pltpu.VMEM_SHARED((NP,), jnp.float32),
    ],
)
def _deg_sc(dst_hbm, ew_hbm, z1_hbm, out_hbm, dst_v, ew_v, acc):
    c = lax.axis_index("c")
    s = lax.axis_index("s")
    b = c * 16 + s
    r0 = pl.multiple_of(s * RPT, 8)
    pltpu.sync_copy(z1_hbm, acc.at[pl.ds(r0, RPT)])
    plsc.subcore_barrier()
    pltpu.sync_copy(dst_hbm.at[b], dst_v)
    pltpu.sync_copy(ew_hbm.at[b], ew_v)

    def chunk(j, carry):
        pltpu.sync_copy(ew_v.at[j], acc.at[dst_v.at[j]], add=True)
        return carry

    lax.fori_loop(0, CHUNKS, chunk, 0)
    plsc.subcore_barrier()
    pltpu.sync_copy(acc.at[pl.ds(r0, RPT)], out_hbm.at[c, pl.ds(r0, RPT)])


@functools.partial(
    pl.kernel,
    mesh=_mesh,
    out_type=jax.ShapeDtypeStruct((2, NP, H), jnp.float32),
    scratch_types=[
        pltpu.VMEM((CHUNKS, K), jnp.int32),
        pltpu.VMEM((CHUNKS, K), jnp.int32),
        pltpu.VMEM((CHUNKS * K,), jnp.float32),
        pltpu.VMEM((K, H), jnp.float32),
        pltpu.VMEM_SHARED((NP, H), jnp.float32),
        pltpu.SemaphoreType.DMA,
    ],
)
def _agg_sc(hp_hbm, src_hbm, dst_hbm, ew_hbm, z2_hbm, out_hbm,
            src_v, dst_v, ew_v, rows_v, acc, sem):
    c = lax.axis_index("c")
    s = lax.axis_index("s")
    b = c * 16 + s
    r0 = pl.multiple_of(s * RPT, 8)
    pltpu.sync_copy(z2_hbm, acc.at[pl.ds(r0, RPT)])
    plsc.subcore_barrier()
    pltpu.sync_copy(src_hbm.at[b], src_v)
    pltpu.sync_copy(dst_hbm.at[b], dst_v)
    pltpu.sync_copy(ew_hbm.at[b], ew_v)

    def chunk(j, carry):
        pltpu.async_copy(hp_hbm.at[src_v.at[j]], rows_v, sem).wait()
        for g in range(8):
            w16 = ew_v[pl.ds(j * K + g * 16, 16)]
            for l in range(16):
                ws = w16.at[jnp.full((16,), l, jnp.int32)].get(
                    mode="promise_in_bounds")
                e = g * 16 + l
                for cg in range(8):
                    sl = pl.ds(cg * 16, 16)
                    rows_v[e, sl] = rows_v[e, sl] * ws
        pltpu.sync_copy(rows_v, acc.at[dst_v.at[j]], add=True)
        return carry

    lax.fori_loop(0, CHUNKS, chunk, 0)
    plsc.subcore_barrier()
    pltpu.sync_copy(acc.at[pl.ds(r0, RPT)], out_hbm.at[c, pl.ds(r0, RPT)])


def _tc1_body(deg0_ref, deg1_ref, x_ref, w_ref, dis_ref, hp_ref):
    deg = 1.0 + deg0_ref[...] + deg1_ref[...]
    dis = jnp.where(deg > 0, lax.rsqrt(deg), 0.0)
    dis_ref[...] = dis
    h = jnp.dot(x_ref[...], w_ref[...], preferred_element_type=jnp.float32,
                precision=lax.Precision.HIGHEST)
    hp_ref[...] = h * dis


def _tc_mid_body(a0_ref, a1_ref, hp_ref, dis_ref, b_ref, w_ref, out_ref):
    dis = dis_ref[...]
    t = (a0_ref[...] + a1_ref[...] + hp_ref[...]) * dis + b_ref[...]
    o = jnp.maximum(t, 0.0)
    out_ref[...] = jnp.dot(o, w_ref[...], preferred_element_type=jnp.float32,
                           precision=lax.Precision.HIGHEST) * dis


def _tc_fin_body(a0_ref, a1_ref, hp_ref, dis_ref, b_ref, brow_ref, wl_ref,
                 bl_ref, out_ref):
    o3 = (a0_ref[...] + a1_ref[...] + hp_ref[...]) * dis_ref[...] + b_ref[...]
    gid = lax.broadcasted_iota(jnp.int32, (G, N), 0)
    oh = (gid == brow_ref[...]).astype(jnp.float32)
    sums = jnp.dot(oh, o3, preferred_element_type=jnp.float32,
                   precision=lax.Precision.HIGHEST)
    cnt = jnp.dot(oh, jnp.ones((N, 1), jnp.float32),
                  preferred_element_type=jnp.float32,
                  precision=lax.Precision.HIGHEST)
    pooled = sums / jnp.maximum(cnt, 1.0)
    out_ref[...] = jnp.dot(pooled, wl_ref[...],
                           preferred_element_type=jnp.float32,
                           precision=lax.Precision.HIGHEST) + bl_ref[...]


def _pc(body, out_shapes):
    return pl.pallas_call(body, out_shape=out_shapes)


def kernel(x, edge_index, edge_attr, batch, W1, b1, W2, b2, W3, b3, Wl, bl):
    pad = EPAD - E
    src3 = jnp.concatenate(
        [edge_index[0], jnp.zeros((pad,), jnp.int32)]).reshape(TILES, CHUNKS, K)
    dst3 = jnp.concatenate(
        [edge_index[1], jnp.zeros((pad,), jnp.int32)]).reshape(TILES, CHUNKS, K)
    ew3 = jnp.concatenate(
        [edge_attr, jnp.zeros((pad,), jnp.float32)]).reshape(TILES, CHUNKS, K)
    ew2 = ew3.reshape(TILES, CHUNKS * K)
    z1 = jnp.zeros((RPT,), jnp.float32)
    z2 = jnp.zeros((RPT, H), jnp.float32)
    brow = batch[None, :]

    deg2 = _deg_sc(dst3, ew3, z1)
    deg0 = deg2[0, :N, None]
    deg1 = deg2[1, :N, None]
    dis, hp1 = _pc(_tc1_body, [
        jax.ShapeDtypeStruct((N, 1), jnp.float32),
        jax.ShapeDtypeStruct((N, H), jnp.float32),
    ])(deg0, deg1, x, W1)

    a1 = _agg_sc(hp1, src3, dst3, ew2, z2)
    hp2 = _pc(_tc_mid_body, jax.ShapeDtypeStruct((N, H), jnp.float32))(
        a1[0, :N], a1[1, :N], hp1, dis, b1[None, :], W2)

    a2 = _agg_sc(hp2, src3, dst3, ew2, z2)
    hp3 = _pc(_tc_mid_body, jax.ShapeDtypeStruct((N, H), jnp.float32))(
        a2[0, :N], a2[1, :N], hp2, dis, b2[None, :], W3)

    a3 = _agg_sc(hp3, src3, dst3, ew2, z2)
    out = _pc(_tc_fin_body, jax.ShapeDtypeStruct((G, C), jnp.float32))(
        a3[0, :N], a3[1, :N], hp3, dis, b3[None, :], brow, Wl, bl[None, :])
    return out
